# baseline (speedup 1.0000x reference)
import jax
import jax.numpy as jnp
from jax.experimental import pallas as pl
from jax.experimental.pallas import tpu as pltpu

_N = 22


def _edge_weight_kernel(a_ref, ew_ref, wk_ref):
    a = a_ref[...]
    deg_row = jnp.sum(a, axis=1, keepdims=True)
    deg_col = jnp.sum(a, axis=0, keepdims=True)
    a_norm = jax.lax.rsqrt(deg_row) * a * jax.lax.rsqrt(deg_col)
    ew = a_norm + jnp.eye(_N, dtype=jnp.float32)
    ew_ref[...] = ew
    wk_ref[...] = jnp.dot(ew, ew, preferred_element_type=jnp.float32)


def _propagate_kernel(p_ref, x_ref, y_ref):
    y_ref[0] = jnp.dot(p_ref[...], x_ref[0],
                       preferred_element_type=jnp.float32)


def kernel(x, edge_weight_param):
    B, C, N, T = x.shape
    assert N == _N

    xs, ys = jnp.tril_indices(N, k=-1)
    a_lower = jnp.zeros((N, N), jnp.float32).at[xs, ys].set(
        edge_weight_param.astype(jnp.float32))
    a_sym = a_lower + a_lower.T + jnp.eye(N, dtype=jnp.float32)

    ew, w2 = pl.pallas_call(
        _edge_weight_kernel,
        out_shape=(
            jax.ShapeDtypeStruct((N, N), jnp.float32),
            jax.ShapeDtypeStruct((N, N), jnp.float32),
        ),
        in_specs=[pl.BlockSpec(memory_space=pltpu.MemorySpace.VMEM)],
        out_specs=(
            pl.BlockSpec(memory_space=pltpu.MemorySpace.VMEM),
            pl.BlockSpec(memory_space=pltpu.MemorySpace.VMEM),
        ),
    )(a_sym)

    p = (w2[:, None, None, :] *
         jnp.eye(C, dtype=jnp.float32)[None, :, :, None]).reshape(N * C, C * N)

    xr = x.reshape(B, C * N, T)
    yr = pl.pallas_call(
        _propagate_kernel,
        out_shape=jax.ShapeDtypeStruct((B, N * C, T), x.dtype),
        grid=(B,),
        in_specs=[
            pl.BlockSpec((N * C, C * N), lambda b: (0, 0)),
            pl.BlockSpec((1, C * N, T), lambda b: (b, 0, 0)),
        ],
        out_specs=pl.BlockSpec((1, N * C, T), lambda b: (b, 0, 0)),
        compiler_params=pltpu.CompilerParams(
            dimension_semantics=("parallel",),
        ),
    )(p, xr)

    return yr.reshape(B, N, C * T), ew


# speedup vs baseline: 1.7942x; 1.7942x over previous
import jax
import jax.numpy as jnp
from jax.experimental import pallas as pl
from jax.experimental.pallas import tpu as pltpu

_N = 22
_BB = 8


def _edge_weight_kernel(a_ref, ew_ref, wk_ref):
    a = a_ref[...]
    deg_row = jnp.sum(a, axis=1, keepdims=True)
    deg_col = jnp.sum(a, axis=0, keepdims=True)
    a_norm = jax.lax.rsqrt(deg_row) * a * jax.lax.rsqrt(deg_col)
    ew = a_norm + jnp.eye(_N, dtype=jnp.float32)
    ew_ref[...] = ew
    wk_ref[...] = jnp.dot(ew, ew, preferred_element_type=jnp.float32)


def _make_propagate(n_batch, n_channels, seq_len):
    def _propagate_kernel(wk_ref, x_ref, y_ref):
        w = wk_ref[...]
        for i in range(n_batch):
            for c in range(n_channels):
                y_ref[i, :, c * seq_len:(c + 1) * seq_len] = jnp.dot(
                    w, x_ref[i, c], preferred_element_type=jnp.float32)
    return _propagate_kernel


def kernel(x, edge_weight_param):
    B, C, N, T = x.shape
    assert N == _N

    xs, ys = jnp.tril_indices(N, k=-1)
    a_lower = jnp.zeros((N, N), jnp.float32).at[xs, ys].set(
        edge_weight_param.astype(jnp.float32))
    a_sym = a_lower + a_lower.T + jnp.eye(N, dtype=jnp.float32)

    ew, w2 = pl.pallas_call(
        _edge_weight_kernel,
        out_shape=(
            jax.ShapeDtypeStruct((N, N), jnp.float32),
            jax.ShapeDtypeStruct((N, N), jnp.float32),
        ),
        in_specs=[pl.BlockSpec(memory_space=pltpu.MemorySpace.VMEM)],
        out_specs=(
            pl.BlockSpec(memory_space=pltpu.MemorySpace.VMEM),
            pl.BlockSpec(memory_space=pltpu.MemorySpace.VMEM),
        ),
    )(a_sym)

    bb = _BB if B % _BB == 0 else 1
    y = pl.pallas_call(
        _make_propagate(bb, C, T),
        out_shape=jax.ShapeDtypeStruct((B, N, C * T), x.dtype),
        grid=(B // bb,),
        in_specs=[
            pl.BlockSpec((N, N), lambda b: (0, 0)),
            pl.BlockSpec((bb, C, N, T), lambda b: (b, 0, 0, 0)),
        ],
        out_specs=pl.BlockSpec((bb, N, C * T), lambda b: (b, 0, 0)),
        compiler_params=pltpu.CompilerParams(
            dimension_semantics=("parallel",),
        ),
    )(w2, x)

    return y, ew


# speedup vs baseline: 2.4686x; 1.3758x over previous
import numpy as np

import jax
import jax.numpy as jnp
from jax.experimental import pallas as pl
from jax.experimental.pallas import tpu as pltpu

_N = 22
_BB = 8
_NL = _N * (_N - 1) // 2

_U = np.zeros((_N, _NL), np.float32)
_V = np.zeros((_NL, _N), np.float32)
for _r in range(1, _N):
    for _q in range(_r):
        _j = _r * (_r - 1) // 2 + _q
        _U[_r, _j] = 1.0
        _V[_j, _q] = 1.0
_UT = np.ascontiguousarray(_U.T)
_VT = np.ascontiguousarray(_V.T)
_EYE = np.eye(_N, dtype=np.float32)


def _make_kernel(n_batch, n_channels, seq_len):
    def _fused_kernel(ewp_ref, u_ref, v_ref, ut_ref, vt_ref, x_ref,
                      y_ref, ew_ref):
        ewp = ewp_ref[...]
        ewp_b = jnp.broadcast_to(ewp, (_N, _NL))
        hi = jax.lax.Precision.HIGHEST
        low = jnp.dot(u_ref[...] * ewp_b, v_ref[...],
                      preferred_element_type=jnp.float32, precision=hi)
        low_t = jnp.dot(vt_ref[...] * ewp_b, ut_ref[...],
                        preferred_element_type=jnp.float32, precision=hi)
        eye = (jax.lax.broadcasted_iota(jnp.int32, (_N, _N), 0) ==
               jax.lax.broadcasted_iota(jnp.int32, (_N, _N), 1)
               ).astype(jnp.float32)
        a = low + low_t + eye
        deg_row = jnp.sum(a, axis=1, keepdims=True)
        deg_col = jnp.sum(a, axis=0, keepdims=True)
        ew = jax.lax.rsqrt(deg_row) * a * jax.lax.rsqrt(deg_col) + eye
        ew_ref[...] = ew
        w2 = jnp.dot(ew, ew, preferred_element_type=jnp.float32)
        for i in range(n_batch):
            for c in range(n_channels):
                y_ref[i, :, c * seq_len:(c + 1) * seq_len] = jnp.dot(
                    w2, x_ref[i, c], preferred_element_type=jnp.float32)
    return _fused_kernel


def kernel(x, edge_weight_param):
    B, C, N, T = x.shape
    assert N == _N

    ewp2d = edge_weight_param.astype(jnp.float32)[None, :]
    bb = _BB if B % _BB == 0 else 1
    const_spec = pl.BlockSpec(lambda b: (0, 0))

    y, ew = pl.pallas_call(
        _make_kernel(bb, C, T),
        out_shape=(
            jax.ShapeDtypeStruct((B, N, C * T), x.dtype),
            jax.ShapeDtypeStruct((N, N), jnp.float32),
        ),
        grid=(B // bb,),
        in_specs=[
            pl.BlockSpec((1, _NL), lambda b: (0, 0)),
            pl.BlockSpec((_N, _NL), lambda b: (0, 0)),
            pl.BlockSpec((_NL, _N), lambda b: (0, 0)),
            pl.BlockSpec((_NL, _N), lambda b: (0, 0)),
            pl.BlockSpec((_N, _NL), lambda b: (0, 0)),
            pl.BlockSpec((bb, C, N, T), lambda b: (b, 0, 0, 0)),
        ],
        out_specs=(
            pl.BlockSpec((bb, N, C * T), lambda b: (b, 0, 0)),
            pl.BlockSpec((N, N), lambda b: (0, 0)),
        ),
        compiler_params=pltpu.CompilerParams(
            dimension_semantics=("parallel",),
        ),
    )(ewp2d, jnp.asarray(_U), jnp.asarray(_V), jnp.asarray(_UT),
      jnp.asarray(_VT), x)

    return y, ew


# speedup vs baseline: 2.5181x; 1.0200x over previous
import numpy as np

import jax
import jax.numpy as jnp
from jax.experimental import pallas as pl
from jax.experimental.pallas import tpu as pltpu

_N = 22
_BB = 16
_NL = _N * (_N - 1) // 2

_U = np.zeros((_N, _NL), np.float32)
_V = np.zeros((_NL, _N), np.float32)
for _r in range(1, _N):
    for _q in range(_r):
        _j = _r * (_r - 1) // 2 + _q
        _U[_r, _j] = 1.0
        _V[_j, _q] = 1.0
_UT = np.ascontiguousarray(_U.T)
_VT = np.ascontiguousarray(_V.T)
_EYE = np.eye(_N, dtype=np.float32)


def _make_kernel(n_batch, n_channels, seq_len):
    def _fused_kernel(ewp_ref, u_ref, v_ref, ut_ref, vt_ref, x_ref,
                      y_ref, ew_ref):
        ewp = ewp_ref[...]
        ewp_b = jnp.broadcast_to(ewp, (_N, _NL))
        hi = jax.lax.Precision.HIGHEST
        low = jnp.dot(u_ref[...] * ewp_b, v_ref[...],
                      preferred_element_type=jnp.float32, precision=hi)
        low_t = jnp.dot(vt_ref[...] * ewp_b, ut_ref[...],
                        preferred_element_type=jnp.float32, precision=hi)
        eye = (jax.lax.broadcasted_iota(jnp.int32, (_N, _N), 0) ==
               jax.lax.broadcasted_iota(jnp.int32, (_N, _N), 1)
               ).astype(jnp.float32)
        a = low + low_t + eye
        deg_row = jnp.sum(a, axis=1, keepdims=True)
        deg_col = jnp.sum(a, axis=0, keepdims=True)
        ew = jax.lax.rsqrt(deg_row) * a * jax.lax.rsqrt(deg_col) + eye
        ew_ref[...] = ew
        w2 = jnp.dot(ew, ew, preferred_element_type=jnp.float32)
        for i in range(n_batch):
            for c in range(n_channels):
                y_ref[i, :, c * seq_len:(c + 1) * seq_len] = jnp.dot(
                    w2, x_ref[i, c], preferred_element_type=jnp.float32)
    return _fused_kernel


def kernel(x, edge_weight_param):
    B, C, N, T = x.shape
    assert N == _N

    ewp2d = edge_weight_param.astype(jnp.float32)[None, :]
    bb = _BB if B % _BB == 0 else 1
    const_spec = pl.BlockSpec(lambda b: (0, 0))

    y, ew = pl.pallas_call(
        _make_kernel(bb, C, T),
        out_shape=(
            jax.ShapeDtypeStruct((B, N, C * T), x.dtype),
            jax.ShapeDtypeStruct((N, N), jnp.float32),
        ),
        grid=(B // bb,),
        in_specs=[
            pl.BlockSpec((1, _NL), lambda b: (0, 0)),
            pl.BlockSpec((_N, _NL), lambda b: (0, 0)),
            pl.BlockSpec((_NL, _N), lambda b: (0, 0)),
            pl.BlockSpec((_NL, _N), lambda b: (0, 0)),
            pl.BlockSpec((_N, _NL), lambda b: (0, 0)),
            pl.BlockSpec((bb, C, N, T), lambda b: (b, 0, 0, 0)),
        ],
        out_specs=(
            pl.BlockSpec((bb, N, C * T), lambda b: (b, 0, 0)),
            pl.BlockSpec((N, N), lambda b: (0, 0)),
        ),
        compiler_params=pltpu.CompilerParams(
            dimension_semantics=("parallel",),
        ),
    )(ewp2d, jnp.asarray(_U), jnp.asarray(_V), jnp.asarray(_UT),
      jnp.asarray(_VT), x)

    return y, ew


# speedup vs baseline: 4.2160x; 1.6743x over previous
import numpy as np

import jax
import jax.numpy as jnp
from jax.experimental import pallas as pl
from jax.experimental.pallas import tpu as pltpu

_N = 22
_BB = 16
_NL = _N * (_N - 1) // 2

_U = np.zeros((_N, _NL), np.float32)
_V = np.zeros((_NL, _N), np.float32)
for _r in range(1, _N):
    for _q in range(_r):
        _j = _r * (_r - 1) // 2 + _q
        _U[_r, _j] = 1.0
        _V[_j, _q] = 1.0
_UT = np.ascontiguousarray(_U.T)
_VT = np.ascontiguousarray(_V.T)


def _make_kernel(n_batch, n_channels, seq_len):
    def _fused_kernel(ewp_ref, u_ref, v_ref, ut_ref, vt_ref, x_ref,
                      y_ref, ew_ref):
        ewp = ewp_ref[...]
        ewp_b = jnp.broadcast_to(ewp, (_N, _NL))
        hi = jax.lax.Precision.HIGHEST
        low = jnp.dot(u_ref[...] * ewp_b, v_ref[...],
                      preferred_element_type=jnp.float32, precision=hi)
        low_t = jnp.dot(vt_ref[...] * ewp_b, ut_ref[...],
                        preferred_element_type=jnp.float32, precision=hi)
        eye = (jax.lax.broadcasted_iota(jnp.int32, (_N, _N), 0) ==
               jax.lax.broadcasted_iota(jnp.int32, (_N, _N), 1)
               ).astype(jnp.float32)
        a = low + low_t + eye
        deg_row = jnp.sum(a, axis=1, keepdims=True)
        deg_col = jnp.sum(a, axis=0, keepdims=True)
        ew = jax.lax.rsqrt(deg_row) * a * jax.lax.rsqrt(deg_col) + eye
        ew_ref[...] = ew
        w2 = jnp.dot(ew, ew, preferred_element_type=jnp.float32)
        for i in range(n_batch):
            for c in range(n_channels):
                y_ref[i, :, c * seq_len:(c + 1) * seq_len] = jnp.dot(
                    w2, x_ref[i, :, c, :], preferred_element_type=jnp.float32)
    return _fused_kernel


def kernel(x, edge_weight_param):
    B, C, N, T = x.shape
    assert N == _N

    ewp2d = edge_weight_param.astype(jnp.float32)[None, :]
    xt = jnp.transpose(x, (0, 2, 1, 3))
    bb = _BB if B % _BB == 0 else 1

    y, ew = pl.pallas_call(
        _make_kernel(bb, C, T),
        out_shape=(
            jax.ShapeDtypeStruct((B, N, C * T), x.dtype),
            jax.ShapeDtypeStruct((N, N), jnp.float32),
        ),
        grid=(B // bb,),
        in_specs=[
            pl.BlockSpec((1, _NL), lambda b: (0, 0)),
            pl.BlockSpec((_N, _NL), lambda b: (0, 0)),
            pl.BlockSpec((_NL, _N), lambda b: (0, 0)),
            pl.BlockSpec((_NL, _N), lambda b: (0, 0)),
            pl.BlockSpec((_N, _NL), lambda b: (0, 0)),
            pl.BlockSpec((bb, N, C, T), lambda b: (b, 0, 0, 0)),
        ],
        out_specs=(
            pl.BlockSpec((bb, N, C * T), lambda b: (b, 0, 0)),
            pl.BlockSpec((N, N), lambda b: (0, 0)),
        ),
        compiler_params=pltpu.CompilerParams(
            dimension_semantics=("parallel",),
        ),
    )(ewp2d, jnp.asarray(_U), jnp.asarray(_V), jnp.asarray(_UT),
      jnp.asarray(_VT), xt)

    return y, ew
